# baseline (speedup 1.0000x reference)
import functools

import numpy as np

import jax
import jax.numpy as jnp
from jax import lax
from jax.experimental import pallas as pl
from jax.experimental.pallas import tpu as pltpu
from jax.experimental.pallas import tpu_sc as plsc


_F32 = jnp.float32
_SEC = 8448
_S2 = 2048
_NT2 = 4

_H1_ORDER = np.concatenate([
    np.arange(1, 33) * 4 - 2,
    np.arange(1, 33) * 4 - 1,
    np.arange(0, 32) * 4,
    np.arange(0, 32) * 4 + 1,
])
_PAD_CHUNKS = (0, _SEC, 2 * _SEC + 8192, 3 * _SEC + 8192)


def _fused_kernel(p1_ref, w1_ref, b1_ref, rw1_ref, rb1_ref,
                  w2_ref, b2_ref, rw2_ref, rb2_ref, cb_ref,
                  bvq_ref, idx_ref, gscr):
    for col in _PAD_CHUNKS:
        gscr[:, col:col + 256] = jnp.zeros((48, 256), _F32)
    for r in range(4):
        p1s = p1_ref[0, :, 8192 * r:8192 * (r + 1)]
        y = jnp.dot(w1_ref[...], p1s, preferred_element_type=_F32) + b1_ref[...]
        y = jax.nn.gelu(y)
        z = jnp.dot(rw1_ref[...], y, preferred_element_type=_F32) + rb1_ref[...]
        g = z[:48] * jax.nn.sigmoid(z[48:])
        off = _SEC * r + (256 if r < 2 else 0)
        gscr[:, off:off + 8192] = g

    cb = cb_ref[...]
    cbsq = jnp.sum(cb * cb, axis=1, keepdims=True)
    for t in range(_NT2):
        pieces = []
        for kh in range(8):
            q, r = kh // 4, kh % 4
            off = _SEC * r + 256 * q + _S2 * t
            pieces.append(gscr[:, off:off + _S2])
        p2 = jnp.concatenate(pieces, axis=0)
        y = jnp.dot(w2_ref[...], p2, preferred_element_type=_F32) + b2_ref[...]
        y = jax.nn.gelu(y)
        z = jnp.dot(rw2_ref[...], y, preferred_element_type=_F32) + rb2_ref[...]
        g = z[:96] * jax.nn.sigmoid(z[96:])
        bvq_ref[0, :, _S2 * t:_S2 * (t + 1)] = g

        scores = jnp.dot(cb, g, preferred_element_type=_F32)
        fsq = jnp.sum(g * g, axis=0, keepdims=True)
        dist = (fsq - 2.0 * scores) + cbsq
        minval = jnp.min(dist, axis=0, keepdims=True)
        kiota = jax.lax.broadcasted_iota(jnp.int32, dist.shape, 0)
        idx = jnp.min(jnp.where(dist == minval, kiota, 1024), axis=0,
                      keepdims=True)
        idx_ref[0, :, _S2 * t:_S2 * (t + 1)] = idx


def _sc_gather(table, idx_flat):
    info = plsc.get_sparse_core_info()
    nc, ns = info.num_cores, info.num_subcores
    nw = nc * ns
    n_tok = idx_flat.shape[0]
    d = table.shape[1]
    per_w = n_tok // nw
    chunk = 512
    n_chunks = per_w // chunk
    mesh = plsc.VectorSubcoreMesh(core_axis_name="c", subcore_axis_name="s")

    @functools.partial(
        pl.kernel, mesh=mesh,
        out_type=jax.ShapeDtypeStruct((n_tok, d), _F32),
        scratch_types=[
            pltpu.VMEM((chunk,), jnp.int32),
            pltpu.VMEM((chunk, d), _F32),
            pltpu.SemaphoreType.DMA,
        ],
    )
    def k(table_hbm, idx_hbm, out_hbm, idx_v, rows_v, sem):
        wid = lax.axis_index("s") * nc + lax.axis_index("c")
        base = wid * per_w
        for c in range(n_chunks):
            lo = base + c * chunk
            pltpu.sync_copy(idx_hbm.at[pl.ds(lo, chunk)], idx_v)
            pltpu.async_copy(table_hbm.at[idx_v], rows_v, sem).wait()
            pltpu.sync_copy(rows_v, out_hbm.at[pl.ds(lo, chunk)])

    return k(table, idx_flat)


def kernel(x, conv1_w, conv1_b, rw1_w, rw1_b, conv2_w, conv2_b, rw2_w, rw2_b,
           codebook):
    B = x.shape[0]
    N1 = 128 * 256
    N2 = 32 * 256

    xp = jnp.pad(x, ((0, 0), (0, 0), (2, 2), (0, 0)))
    p1 = jnp.stack([xp[:, :, kh:kh + 509:4, :] for kh in range(8)], axis=1)
    p1 = p1[:, :, :, _H1_ORDER, :].reshape(B, 16, N1)
    w1m = jnp.transpose(conv1_w[:, :, :, 0], (0, 2, 1)).reshape(48, 16)
    w2m = jnp.transpose(conv2_w[:, :, :, 0], (0, 2, 1)).reshape(96, 384)

    full = lambda *s: pl.BlockSpec(s, lambda b: tuple(0 for _ in s))
    bvq, idx = pl.pallas_call(
        _fused_kernel,
        grid=(B,),
        in_specs=[
            pl.BlockSpec((1, 16, N1), lambda b: (b, 0, 0)),
            full(48, 16), full(48, 1), full(96, 48), full(96, 1),
            full(96, 384), full(96, 1), full(192, 96), full(192, 1),
            full(1024, 96),
        ],
        out_specs=[
            pl.BlockSpec((1, 96, N2), lambda b: (b, 0, 0)),
            pl.BlockSpec((1, 1, N2), lambda b: (b, 0, 0)),
        ],
        out_shape=[
            jax.ShapeDtypeStruct((B, 96, N2), _F32),
            jax.ShapeDtypeStruct((B, 1, N2), jnp.int32),
        ],
        scratch_shapes=[pltpu.VMEM((48, 4 * _SEC), _F32)],
    )(p1, w1m, conv1_b[:, None], rw1_w[:, :, 0, 0], rw1_b[:, None],
      w2m, conv2_b[:, None], rw2_w[:, :, 0, 0], rw2_b[:, None], codebook)

    indices = idx.reshape(B, N2)
    cb_pad = jnp.pad(codebook, ((0, 0), (0, 32)))
    quant = _sc_gather(cb_pad, indices.reshape(B * N2))[:, :96]
    latent = jnp.transpose(quant.reshape(B, N2, 96), (0, 2, 1))
    latent = latent.reshape(B, 96, 32, 256)
    beforvq = bvq.reshape(B, 96, 32, 256)
    return (latent, indices, beforvq)


# speedup vs baseline: 6.1068x; 6.1068x over previous
import functools

import numpy as np

import jax
import jax.numpy as jnp
from jax import lax
from jax.experimental import pallas as pl
from jax.experimental.pallas import tpu as pltpu
from jax.experimental.pallas import tpu_sc as plsc


_F32 = jnp.float32
_SEC = 8448
_S2 = 2048
_NT2 = 4

_H1_ORDER = np.concatenate([
    np.arange(1, 33) * 4 - 2,
    np.arange(1, 33) * 4 - 1,
    np.arange(0, 32) * 4,
    np.arange(0, 32) * 4 + 1,
])
_PAD_CHUNKS = (0, _SEC, 2 * _SEC + 8192, 3 * _SEC + 8192)


def _fused_kernel(p1_ref, w1_ref, b1_ref, rw1_ref, rb1_ref,
                  w2_ref, b2_ref, rw2_ref, rb2_ref, cb_ref,
                  bvq_ref, idx_ref, gscr):
    for col in _PAD_CHUNKS:
        gscr[:, col:col + 256] = jnp.zeros((48, 256), _F32)
    for r in range(4):
        p1s = p1_ref[0, :, 8192 * r:8192 * (r + 1)]
        y = jnp.dot(w1_ref[...], p1s, preferred_element_type=_F32) + b1_ref[...]
        y = jax.nn.gelu(y)
        z = jnp.dot(rw1_ref[...], y, preferred_element_type=_F32) + rb1_ref[...]
        g = z[:48] * jax.nn.sigmoid(z[48:])
        off = _SEC * r + (256 if r < 2 else 0)
        gscr[:, off:off + 8192] = g

    cb = cb_ref[...]
    cbsq = jnp.sum(cb * cb, axis=1, keepdims=True)
    for t in range(_NT2):
        pieces = []
        for kh in range(8):
            q, r = kh // 4, kh % 4
            off = _SEC * r + 256 * q + _S2 * t
            pieces.append(gscr[:, off:off + _S2])
        p2 = jnp.concatenate(pieces, axis=0)
        y = jnp.dot(w2_ref[...], p2, preferred_element_type=_F32) + b2_ref[...]
        y = jax.nn.gelu(y)
        z = jnp.dot(rw2_ref[...], y, preferred_element_type=_F32) + rb2_ref[...]
        g = z[:96] * jax.nn.sigmoid(z[96:])
        bvq_ref[0, :, _S2 * t:_S2 * (t + 1)] = g

        scores = jnp.dot(cb, g, preferred_element_type=_F32)
        fsq = jnp.sum(g * g, axis=0, keepdims=True)
        dist = (fsq - 2.0 * scores) + cbsq
        minval = jnp.min(dist, axis=0, keepdims=True)
        kiota = jax.lax.broadcasted_iota(jnp.int32, dist.shape, 0)
        idx = jnp.min(jnp.where(dist == minval, kiota, 1024), axis=0,
                      keepdims=True)
        idx_ref[0, :, _S2 * t:_S2 * (t + 1)] = idx


def _sc_gather(table, idx_flat):
    info = plsc.get_sparse_core_info()
    nc, ns = info.num_cores, info.num_subcores
    nw = nc * ns
    n_tok = idx_flat.shape[0]
    d = table.shape[1]
    per_w = n_tok // nw
    chunk = 512
    n_chunks = per_w // chunk
    mesh = plsc.VectorSubcoreMesh(core_axis_name="c", subcore_axis_name="s")

    @functools.partial(
        pl.kernel, mesh=mesh,
        out_type=jax.ShapeDtypeStruct((n_tok, d), _F32),
        scratch_types=[
            pltpu.VMEM((chunk,), jnp.int32),
            pltpu.VMEM((chunk, d), _F32),
            pltpu.VMEM_SHARED((table.shape[0], d), _F32),
            pltpu.SemaphoreType.DMA,
        ],
    )
    def k(table_hbm, idx_hbm, out_hbm, idx_v, rows_v, tab_sh, sem):
        sid = lax.axis_index("s")
        @pl.when(sid == 0)
        def _():
            pltpu.sync_copy(table_hbm, tab_sh)
        plsc.subcore_barrier()
        wid = sid * nc + lax.axis_index("c")
        base = wid * per_w
        for c in range(n_chunks):
            lo = base + c * chunk
            pltpu.sync_copy(idx_hbm.at[pl.ds(lo, chunk)], idx_v)
            pltpu.async_copy(tab_sh.at[idx_v], rows_v, sem).wait()
            pltpu.sync_copy(rows_v, out_hbm.at[pl.ds(lo, chunk)])

    return k(table, idx_flat)


def kernel(x, conv1_w, conv1_b, rw1_w, rw1_b, conv2_w, conv2_b, rw2_w, rw2_b,
           codebook):
    B = x.shape[0]
    N1 = 128 * 256
    N2 = 32 * 256

    xp = jnp.pad(x, ((0, 0), (0, 0), (2, 2), (0, 0)))
    p1 = jnp.stack([xp[:, :, kh:kh + 509:4, :] for kh in range(8)], axis=1)
    p1 = p1[:, :, :, _H1_ORDER, :].reshape(B, 16, N1)
    w1m = jnp.transpose(conv1_w[:, :, :, 0], (0, 2, 1)).reshape(48, 16)
    w2m = jnp.transpose(conv2_w[:, :, :, 0], (0, 2, 1)).reshape(96, 384)

    full = lambda *s: pl.BlockSpec(s, lambda b: tuple(0 for _ in s))
    bvq, idx = pl.pallas_call(
        _fused_kernel,
        grid=(B,),
        in_specs=[
            pl.BlockSpec((1, 16, N1), lambda b: (b, 0, 0)),
            full(48, 16), full(48, 1), full(96, 48), full(96, 1),
            full(96, 384), full(96, 1), full(192, 96), full(192, 1),
            full(1024, 96),
        ],
        out_specs=[
            pl.BlockSpec((1, 96, N2), lambda b: (b, 0, 0)),
            pl.BlockSpec((1, 1, N2), lambda b: (b, 0, 0)),
        ],
        out_shape=[
            jax.ShapeDtypeStruct((B, 96, N2), _F32),
            jax.ShapeDtypeStruct((B, 1, N2), jnp.int32),
        ],
        scratch_shapes=[pltpu.VMEM((48, 4 * _SEC), _F32)],
    )(p1, w1m, conv1_b[:, None], rw1_w[:, :, 0, 0], rw1_b[:, None],
      w2m, conv2_b[:, None], rw2_w[:, :, 0, 0], rw2_b[:, None], codebook)

    indices = idx.reshape(B, N2)
    cb_pad = jnp.pad(codebook, ((0, 0), (0, 32)))
    quant = _sc_gather(cb_pad, indices.reshape(B * N2))[:, :96]
    latent = jnp.transpose(quant.reshape(B, N2, 96), (0, 2, 1))
    latent = latent.reshape(B, 96, 32, 256)
    beforvq = bvq.reshape(B, 96, 32, 256)
    return (latent, indices, beforvq)


# speedup vs baseline: 6.4326x; 1.0533x over previous
import functools

import numpy as np

import jax
import jax.numpy as jnp
from jax import lax
from jax.experimental import pallas as pl
from jax.experimental.pallas import tpu as pltpu
from jax.experimental.pallas import tpu_sc as plsc


_F32 = jnp.float32
_SEC = 8448
_S2 = 2048
_NT2 = 4

_H1_ORDER = np.concatenate([
    np.arange(1, 33) * 4 - 2,
    np.arange(1, 33) * 4 - 1,
    np.arange(0, 32) * 4,
    np.arange(0, 32) * 4 + 1,
])
_PAD_CHUNKS = (0, _SEC, 2 * _SEC + 8192, 3 * _SEC + 8192)


def _fused_kernel(p1_ref, w1_ref, b1_ref, rw1_ref, rb1_ref,
                  w2_ref, b2_ref, rw2_ref, rb2_ref, cb_ref,
                  bvq_ref, idx_ref, gscr):
    for col in _PAD_CHUNKS:
        gscr[:, col:col + 256] = jnp.zeros((48, 256), _F32)
    for r in range(4):
        p1s = p1_ref[0, :, 8192 * r:8192 * (r + 1)]
        y = jnp.dot(w1_ref[...], p1s, preferred_element_type=_F32) + b1_ref[...]
        y = jax.nn.gelu(y)
        z = jnp.dot(rw1_ref[...], y, preferred_element_type=_F32) + rb1_ref[...]
        g = z[:48] * jax.nn.sigmoid(z[48:])
        off = _SEC * r + (256 if r < 2 else 0)
        gscr[:, off:off + 8192] = g

    cb = cb_ref[...]
    cbsq = jnp.sum(cb * cb, axis=1, keepdims=True)
    for t in range(_NT2):
        pieces = []
        for kh in range(8):
            q, r = kh // 4, kh % 4
            off = _SEC * r + 256 * q + _S2 * t
            pieces.append(gscr[:, off:off + _S2])
        p2 = jnp.concatenate(pieces, axis=0)
        y = jnp.dot(w2_ref[...], p2, preferred_element_type=_F32) + b2_ref[...]
        y = jax.nn.gelu(y)
        z = jnp.dot(rw2_ref[...], y, preferred_element_type=_F32) + rb2_ref[...]
        g = z[:96] * jax.nn.sigmoid(z[96:])
        bvq_ref[0, :, _S2 * t:_S2 * (t + 1)] = g

        scores = jnp.dot(cb, g, preferred_element_type=_F32)
        fsq = jnp.sum(g * g, axis=0, keepdims=True)
        dist = (fsq - 2.0 * scores) + cbsq
        minval = jnp.min(dist, axis=0, keepdims=True)
        kiota = jax.lax.broadcasted_iota(jnp.int32, dist.shape, 0)
        idx = jnp.min(jnp.where(dist == minval, kiota, 1024), axis=0,
                      keepdims=True)
        idx_ref[0, :, _S2 * t:_S2 * (t + 1)] = idx


def _sc_gather(table, idx_flat):
    info = plsc.get_sparse_core_info()
    nc, ns = info.num_cores, info.num_subcores
    nw = nc * ns
    n_tok = idx_flat.shape[0]
    d = table.shape[1]
    per_w = n_tok // nw
    chunk = 512
    n_chunks = per_w // chunk
    mesh = plsc.VectorSubcoreMesh(core_axis_name="c", subcore_axis_name="s")

    @functools.partial(
        pl.kernel, mesh=mesh,
        out_type=jax.ShapeDtypeStruct((n_tok, d), _F32),
        scratch_types=[
            pltpu.VMEM((chunk,), jnp.int32),
            pltpu.VMEM((chunk, d), _F32),
            pltpu.VMEM_SHARED((table.shape[0], d), _F32),
            pltpu.SemaphoreType.DMA,
        ],
    )
    def k(table_hbm, idx_hbm, out_hbm, idx_v, rows_v, tab_sh, sem):
        sid = lax.axis_index("s")
        @pl.when(sid == 0)
        def _():
            pltpu.sync_copy(table_hbm, tab_sh)
        plsc.subcore_barrier()
        wid = sid * nc + lax.axis_index("c")
        base = wid * per_w
        for c in range(n_chunks):
            lo = base + c * chunk
            pltpu.sync_copy(idx_hbm.at[pl.ds(lo, chunk)], idx_v)
            pltpu.async_copy(tab_sh.at[idx_v], rows_v, sem).wait()
            pltpu.sync_copy(rows_v, out_hbm.at[pl.ds(lo, chunk)])

    return k(table, idx_flat)


def _encode_half(xh, w1m, b1, rw1m, rb1, w2m, b2, rw2m, rb2, codebook):
    Bh = xh.shape[0]
    N1 = 128 * 256
    N2 = 32 * 256
    xp = jnp.pad(xh, ((0, 0), (0, 0), (2, 2), (0, 0)))
    p1 = jnp.stack([xp[:, :, kh:kh + 509:4, :] for kh in range(8)], axis=1)
    p1 = p1[:, :, :, _H1_ORDER, :].reshape(Bh, 16, N1)

    full = lambda *s: pl.BlockSpec(s, lambda b: tuple(0 for _ in s))
    bvq, idx = pl.pallas_call(
        _fused_kernel,
        grid=(Bh,),
        in_specs=[
            pl.BlockSpec((1, 16, N1), lambda b: (b, 0, 0)),
            full(48, 16), full(48, 1), full(96, 48), full(96, 1),
            full(96, 384), full(96, 1), full(192, 96), full(192, 1),
            full(1024, 96),
        ],
        out_specs=[
            pl.BlockSpec((1, 96, N2), lambda b: (b, 0, 0)),
            pl.BlockSpec((1, 1, N2), lambda b: (b, 0, 0)),
        ],
        out_shape=[
            jax.ShapeDtypeStruct((Bh, 96, N2), _F32),
            jax.ShapeDtypeStruct((Bh, 1, N2), jnp.int32),
        ],
        scratch_shapes=[pltpu.VMEM((48, 4 * _SEC), _F32)],
    )(p1, w1m, b1, rw1m, rb1, w2m, b2, rw2m, rb2, codebook)
    return bvq, idx.reshape(Bh, N2)


def kernel(x, conv1_w, conv1_b, rw1_w, rw1_b, conv2_w, conv2_b, rw2_w, rw2_b,
           codebook):
    B = x.shape[0]
    Bh = B // 2
    N2 = 32 * 256

    w1m = jnp.transpose(conv1_w[:, :, :, 0], (0, 2, 1)).reshape(48, 16)
    w2m = jnp.transpose(conv2_w[:, :, :, 0], (0, 2, 1)).reshape(96, 384)
    args = (w1m, conv1_b[:, None], rw1_w[:, :, 0, 0], rw1_b[:, None],
            w2m, conv2_b[:, None], rw2_w[:, :, 0, 0], rw2_b[:, None], codebook)
    cb_pad = jnp.pad(codebook, ((0, 0), (0, 32)))

    halves = []
    for h in range(2):
        bvq_h, idx_h = _encode_half(x[h * Bh:(h + 1) * Bh], *args)
        quant_h = _sc_gather(cb_pad, idx_h.reshape(Bh * N2))[:, :96]
        lat_h = jnp.transpose(quant_h.reshape(Bh, N2, 96), (0, 2, 1))
        halves.append((lat_h, idx_h, bvq_h))

    latent = jnp.concatenate([h[0] for h in halves]).reshape(B, 96, 32, 256)
    indices = jnp.concatenate([h[1] for h in halves])
    beforvq = jnp.concatenate([h[2] for h in halves]).reshape(B, 96, 32, 256)
    return (latent, indices, beforvq)


# speedup vs baseline: 6.9075x; 1.0738x over previous
import functools

import numpy as np

import jax
import jax.numpy as jnp
from jax import lax
from jax.experimental import pallas as pl
from jax.experimental.pallas import tpu as pltpu
from jax.experimental.pallas import tpu_sc as plsc


_F32 = jnp.float32
_SEC = 8448
_S2 = 2048
_NT2 = 4

_H1_ORDER = np.concatenate([
    np.arange(1, 33) * 4 - 2,
    np.arange(1, 33) * 4 - 1,
    np.arange(0, 32) * 4,
    np.arange(0, 32) * 4 + 1,
])
_PAD_CHUNKS = (0, _SEC, 2 * _SEC + 8192, 3 * _SEC + 8192)


def _fused_kernel(p1_ref, w1_ref, b1_ref, rw1_ref, rb1_ref,
                  w2_ref, b2_ref, rw2_ref, rb2_ref, cb_ref, kio_ref,
                  bvq_ref, idx_ref, gscr):
    for col in _PAD_CHUNKS:
        gscr[:, col:col + 256] = jnp.zeros((48, 256), _F32)
    for r in range(4):
        p1s = p1_ref[0, :, 8192 * r:8192 * (r + 1)]
        y = jnp.dot(w1_ref[...], p1s, preferred_element_type=_F32) + b1_ref[...]
        y = jax.nn.gelu(y)
        z = jnp.dot(rw1_ref[...], y, preferred_element_type=_F32) + rb1_ref[...]
        g = z[:48] * jax.nn.sigmoid(z[48:])
        off = _SEC * r + (256 if r < 2 else 0)
        gscr[:, off:off + 8192] = g

    cb = cb_ref[...]
    cbsq = jnp.sum(cb * cb, axis=1, keepdims=True)
    a_aug = jnp.concatenate([cb * -2.0, cbsq], axis=1)
    for t in range(_NT2):
        pieces = []
        for kh in range(8):
            q, r = kh // 4, kh % 4
            off = _SEC * r + 256 * q + _S2 * t
            pieces.append(gscr[:, off:off + _S2])
        p2 = jnp.concatenate(pieces, axis=0)
        y = jnp.dot(w2_ref[...], p2, preferred_element_type=_F32) + b2_ref[...]
        y = jax.nn.gelu(y)
        z = jnp.dot(rw2_ref[...], y, preferred_element_type=_F32) + rb2_ref[...]
        g = z[:96] * jax.nn.sigmoid(z[96:])
        bvq_ref[0, :, _S2 * t:_S2 * (t + 1)] = g

        g_aug = jnp.concatenate([g, jnp.ones((1, _S2), _F32)], axis=0)
        dist = jnp.dot(a_aug, g_aug, preferred_element_type=_F32)
        minval = jnp.min(dist, axis=0, keepdims=True)
        idxf = jnp.min(jnp.where(dist == minval, kio_ref[...], 2048.0), axis=0,
                       keepdims=True)
        idx_ref[0, :, _S2 * t:_S2 * (t + 1)] = idxf.astype(jnp.int32)


def _sc_gather(table, idx_flat):
    info = plsc.get_sparse_core_info()
    nc, ns = info.num_cores, info.num_subcores
    nw = nc * ns
    n_tok = idx_flat.shape[0]
    d = table.shape[1]
    per_w = n_tok // nw
    chunk = 512
    n_chunks = per_w // chunk
    mesh = plsc.VectorSubcoreMesh(core_axis_name="c", subcore_axis_name="s")

    @functools.partial(
        pl.kernel, mesh=mesh,
        out_type=jax.ShapeDtypeStruct((n_tok, d), _F32),
        scratch_types=[
            pltpu.VMEM((chunk,), jnp.int32),
            pltpu.VMEM((chunk, d), _F32),
            pltpu.VMEM_SHARED((table.shape[0], d), _F32),
            pltpu.SemaphoreType.DMA,
        ],
    )
    def k(table_hbm, idx_hbm, out_hbm, idx_v, rows_v, tab_sh, sem):
        sid = lax.axis_index("s")
        @pl.when(sid == 0)
        def _():
            pltpu.sync_copy(table_hbm, tab_sh)
        plsc.subcore_barrier()
        wid = sid * nc + lax.axis_index("c")
        base = wid * per_w
        for c in range(n_chunks):
            lo = base + c * chunk
            pltpu.sync_copy(idx_hbm.at[pl.ds(lo, chunk)], idx_v)
            pltpu.async_copy(tab_sh.at[idx_v], rows_v, sem).wait()
            pltpu.sync_copy(rows_v, out_hbm.at[pl.ds(lo, chunk)])

    return k(table, idx_flat)


def _encode_half(xh, w1m, b1, rw1m, rb1, w2m, b2, rw2m, rb2, codebook):
    Bh = xh.shape[0]
    N1 = 128 * 256
    N2 = 32 * 256
    xp = jnp.pad(xh, ((0, 0), (0, 0), (2, 2), (0, 0)))
    p1 = jnp.stack([xp[:, :, kh:kh + 509:4, :] for kh in range(8)], axis=1)
    p1 = p1[:, :, :, _H1_ORDER, :].reshape(Bh, 16, N1)

    full = lambda *s: pl.BlockSpec(s, lambda b: tuple(0 for _ in s))
    bvq, idx = pl.pallas_call(
        _fused_kernel,
        grid=(Bh,),
        in_specs=[
            pl.BlockSpec((1, 16, N1), lambda b: (b, 0, 0)),
            full(48, 16), full(48, 1), full(96, 48), full(96, 1),
            full(96, 384), full(96, 1), full(192, 96), full(192, 1),
            full(1024, 96), full(1024, 1),
        ],
        out_specs=[
            pl.BlockSpec((1, 96, N2), lambda b: (b, 0, 0)),
            pl.BlockSpec((1, 1, N2), lambda b: (b, 0, 0)),
        ],
        out_shape=[
            jax.ShapeDtypeStruct((Bh, 96, N2), _F32),
            jax.ShapeDtypeStruct((Bh, 1, N2), jnp.int32),
        ],
        scratch_shapes=[pltpu.VMEM((48, 4 * _SEC), _F32)],
    )(p1, w1m, b1, rw1m, rb1, w2m, b2, rw2m, rb2, codebook,
      jnp.arange(1024, dtype=_F32)[:, None])
    return bvq, idx.reshape(Bh, N2)


def kernel(x, conv1_w, conv1_b, rw1_w, rw1_b, conv2_w, conv2_b, rw2_w, rw2_b,
           codebook):
    B = x.shape[0]
    Bh = B // 2
    N2 = 32 * 256

    w1m = jnp.transpose(conv1_w[:, :, :, 0], (0, 2, 1)).reshape(48, 16)
    w2m = jnp.transpose(conv2_w[:, :, :, 0], (0, 2, 1)).reshape(96, 384)
    args = (w1m, conv1_b[:, None], rw1_w[:, :, 0, 0], rw1_b[:, None],
            w2m, conv2_b[:, None], rw2_w[:, :, 0, 0], rw2_b[:, None], codebook)
    cb_pad = jnp.pad(codebook, ((0, 0), (0, 32)))

    halves = []
    for h in range(2):
        bvq_h, idx_h = _encode_half(x[h * Bh:(h + 1) * Bh], *args)
        quant_h = _sc_gather(cb_pad, idx_h.reshape(Bh * N2))[:, :96]
        lat_h = jnp.transpose(quant_h.reshape(Bh, N2, 96), (0, 2, 1))
        halves.append((lat_h, idx_h, bvq_h))

    latent = jnp.concatenate([h[0] for h in halves]).reshape(B, 96, 32, 256)
    indices = jnp.concatenate([h[1] for h in halves])
    beforvq = jnp.concatenate([h[2] for h in halves]).reshape(B, 96, 32, 256)
    return (latent, indices, beforvq)


# speedup vs baseline: 6.9146x; 1.0010x over previous
import functools

import numpy as np

import jax
import jax.numpy as jnp
from jax import lax
from jax.experimental import pallas as pl
from jax.experimental.pallas import tpu as pltpu
from jax.experimental.pallas import tpu_sc as plsc


_F32 = jnp.float32
_SEC = 8448
_S2 = 2048
_NT2 = 4

_H1_ORDER = np.concatenate([
    np.arange(1, 33) * 4 - 2,
    np.arange(1, 33) * 4 - 1,
    np.arange(0, 32) * 4,
    np.arange(0, 32) * 4 + 1,
])
_PAD_CHUNKS = (0, _SEC, 2 * _SEC + 8192, 3 * _SEC + 8192)


def _fused_kernel(p1_ref, w1_ref, b1_ref, rw1_ref, rb1_ref,
                  w2_ref, b2_ref, rw2_ref, rb2_ref, cb_ref, kio_ref,
                  bvq_ref, idx_ref, gscr):
    for col in _PAD_CHUNKS:
        gscr[:, col:col + 256] = jnp.zeros((48, 256), _F32)
    for r in range(4):
        p1s = p1_ref[0, :, 8192 * r:8192 * (r + 1)]
        y = jnp.dot(w1_ref[...], p1s, preferred_element_type=_F32) + b1_ref[...]
        y = jax.nn.gelu(y)
        z = jnp.dot(rw1_ref[...], y, preferred_element_type=_F32) + rb1_ref[...]
        g = z[:48] * jax.nn.sigmoid(z[48:])
        off = _SEC * r + (256 if r < 2 else 0)
        gscr[:, off:off + 8192] = g

    cb = cb_ref[...]
    cbsq = jnp.sum(cb * cb, axis=1, keepdims=True)
    a_aug = jnp.concatenate([cb * -2.0, cbsq], axis=1)
    for t in range(_NT2):
        pieces = []
        for kh in range(8):
            q, r = kh // 4, kh % 4
            off = _SEC * r + 256 * q + _S2 * t
            pieces.append(gscr[:, off:off + _S2])
        p2 = jnp.concatenate(pieces, axis=0)
        y = jnp.dot(w2_ref[...], p2, preferred_element_type=_F32) + b2_ref[...]
        y = jax.nn.gelu(y)
        z = jnp.dot(rw2_ref[...], y, preferred_element_type=_F32) + rb2_ref[...]
        g = z[:96] * jax.nn.sigmoid(z[96:])
        bvq_ref[0, :, _S2 * t:_S2 * (t + 1)] = g

        g_aug = jnp.concatenate([g, jnp.ones((1, _S2), _F32)], axis=0)
        dist = jnp.dot(a_aug, g_aug, preferred_element_type=_F32)
        minval = jnp.min(dist, axis=0, keepdims=True)
        idxf = jnp.min(jnp.where(dist == minval, kio_ref[...], 2048.0), axis=0,
                       keepdims=True)
        idx_ref[0, :, _S2 * t:_S2 * (t + 1)] = idxf.astype(jnp.int32)


def _sc_gather(table, idx_flat):
    info = plsc.get_sparse_core_info()
    nc, ns = info.num_cores, info.num_subcores
    nw = nc * ns
    n_tok = idx_flat.shape[0]
    d = table.shape[1]
    per_w = n_tok // nw
    chunk = 512
    n_chunks = per_w // chunk
    mesh = plsc.VectorSubcoreMesh(core_axis_name="c", subcore_axis_name="s")

    @functools.partial(
        pl.kernel, mesh=mesh,
        out_type=jax.ShapeDtypeStruct((n_tok, d), _F32),
        scratch_types=[
            pltpu.VMEM((chunk,), jnp.int32),
            pltpu.VMEM((chunk, d), _F32),
            pltpu.VMEM_SHARED((table.shape[0], d), _F32),
            pltpu.SemaphoreType.DMA,
        ],
    )
    def k(table_hbm, idx_hbm, out_hbm, idx_v, rows_v, tab_sh, sem):
        sid = lax.axis_index("s")
        @pl.when(sid == 0)
        def _():
            pltpu.sync_copy(table_hbm, tab_sh)
        plsc.subcore_barrier()
        wid = sid * nc + lax.axis_index("c")
        base = wid * per_w
        for c in range(n_chunks):
            lo = base + c * chunk
            pltpu.sync_copy(idx_hbm.at[pl.ds(lo, chunk)], idx_v)
            pltpu.async_copy(tab_sh.at[idx_v], rows_v, sem).wait()
            pltpu.sync_copy(rows_v, out_hbm.at[pl.ds(lo, chunk)])

    return k(table, idx_flat)


def _encode_half(xh, w1m, b1, rw1m, rb1, w2m, b2, rw2m, rb2, codebook):
    Bh = xh.shape[0]
    N1 = 128 * 256
    N2 = 32 * 256
    xp = jnp.pad(xh, ((0, 0), (0, 0), (2, 2), (0, 0)))
    p1 = jnp.stack([xp[:, :, kh:kh + 509:4, :] for kh in range(8)], axis=1)
    p1 = p1[:, :, :, _H1_ORDER, :].reshape(Bh, 16, N1)

    full = lambda *s: pl.BlockSpec(s, lambda b: tuple(0 for _ in s))
    bvq, idx = pl.pallas_call(
        _fused_kernel,
        grid=(Bh,),
        in_specs=[
            pl.BlockSpec((1, 16, N1), lambda b: (b, 0, 0)),
            full(48, 16), full(48, 1), full(96, 48), full(96, 1),
            full(96, 384), full(96, 1), full(192, 96), full(192, 1),
            full(1024, 96), full(1024, 1),
        ],
        out_specs=[
            pl.BlockSpec((1, 96, N2), lambda b: (b, 0, 0)),
            pl.BlockSpec((1, 1, N2), lambda b: (b, 0, 0)),
        ],
        out_shape=[
            jax.ShapeDtypeStruct((Bh, 96, N2), _F32),
            jax.ShapeDtypeStruct((Bh, 1, N2), jnp.int32),
        ],
        scratch_shapes=[pltpu.VMEM((48, 4 * _SEC), _F32)],
    )(p1, w1m, b1, rw1m, rb1, w2m, b2, rw2m, rb2, codebook,
      jnp.arange(1024, dtype=_F32)[:, None])
    return bvq, idx.reshape(Bh, N2)


def kernel(x, conv1_w, conv1_b, rw1_w, rw1_b, conv2_w, conv2_b, rw2_w, rw2_b,
           codebook):
    B = x.shape[0]
    NSPLIT = 4
    Bh = B // NSPLIT
    N2 = 32 * 256

    w1m = jnp.transpose(conv1_w[:, :, :, 0], (0, 2, 1)).reshape(48, 16)
    w2m = jnp.transpose(conv2_w[:, :, :, 0], (0, 2, 1)).reshape(96, 384)
    args = (w1m, conv1_b[:, None], rw1_w[:, :, 0, 0], rw1_b[:, None],
            w2m, conv2_b[:, None], rw2_w[:, :, 0, 0], rw2_b[:, None], codebook)
    cb_pad = jnp.pad(codebook, ((0, 0), (0, 32)))

    halves = []
    for h in range(NSPLIT):
        bvq_h, idx_h = _encode_half(x[h * Bh:(h + 1) * Bh], *args)
        quant_h = _sc_gather(cb_pad, idx_h.reshape(Bh * N2))[:, :96]
        lat_h = jnp.transpose(quant_h.reshape(Bh, N2, 96), (0, 2, 1))
        halves.append((lat_h, idx_h, bvq_h))

    latent = jnp.concatenate([h[0] for h in halves]).reshape(B, 96, 32, 256)
    indices = jnp.concatenate([h[1] for h in halves])
    beforvq = jnp.concatenate([h[2] for h in halves]).reshape(B, 96, 32, 256)
    return (latent, indices, beforvq)


# speedup vs baseline: 6.9459x; 1.0045x over previous
import functools

import numpy as np

import jax
import jax.numpy as jnp
from jax import lax
from jax.experimental import pallas as pl
from jax.experimental.pallas import tpu as pltpu
from jax.experimental.pallas import tpu_sc as plsc


_F32 = jnp.float32
_SEC = 8448
_S2 = 2048
_NT2 = 4

_H1_ORDER = np.concatenate([
    np.arange(1, 33) * 4 - 2,
    np.arange(1, 33) * 4 - 1,
    np.arange(0, 32) * 4,
    np.arange(0, 32) * 4 + 1,
])
_PAD_CHUNKS = (0, _SEC, 2 * _SEC + 8192, 3 * _SEC + 8192)


def _fused_kernel(p1_ref, w1_ref, b1_ref, rw1_ref, rb1_ref,
                  w2_ref, b2_ref, rw2_ref, rb2_ref, cb_ref, kio_ref,
                  bvq_ref, idx_ref, gscr):
    for col in _PAD_CHUNKS:
        gscr[:, col:col + 256] = jnp.zeros((48, 256), _F32)
    p1s = p1_ref[0]
    y = jnp.dot(w1_ref[...], p1s, preferred_element_type=_F32) + b1_ref[...]
    y = jax.nn.gelu(y)
    z = jnp.dot(rw1_ref[...], y, preferred_element_type=_F32) + rb1_ref[...]
    g1 = z[:48] * jax.nn.sigmoid(z[48:])
    for r in range(4):
        off = _SEC * r + (256 if r < 2 else 0)
        gscr[:, off:off + 8192] = g1[:, 8192 * r:8192 * (r + 1)]

    cb = cb_ref[...]
    cbsq = jnp.sum(cb * cb, axis=1, keepdims=True)
    a_aug = jnp.concatenate([cb * -2.0, cbsq], axis=1)
    for t in range(_NT2):
        pieces = []
        for kh in range(8):
            q, r = kh // 4, kh % 4
            off = _SEC * r + 256 * q + _S2 * t
            pieces.append(gscr[:, off:off + _S2])
        p2 = jnp.concatenate(pieces, axis=0)
        y = jnp.dot(w2_ref[...], p2, preferred_element_type=_F32) + b2_ref[...]
        y = jax.nn.gelu(y)
        z = jnp.dot(rw2_ref[...], y, preferred_element_type=_F32) + rb2_ref[...]
        g = z[:96] * jax.nn.sigmoid(z[96:])
        bvq_ref[0, :, _S2 * t:_S2 * (t + 1)] = g

        g_aug = jnp.concatenate([g, jnp.ones((1, _S2), _F32)], axis=0)
        dist = jnp.dot(a_aug, g_aug, preferred_element_type=_F32)
        minval = jnp.min(dist, axis=0, keepdims=True)
        idxf = jnp.min(jnp.where(dist == minval, kio_ref[...], 2048.0), axis=0,
                       keepdims=True)
        idx_ref[0, :, _S2 * t:_S2 * (t + 1)] = idxf.astype(jnp.int32)


def _sc_gather(table, idx_flat):
    info = plsc.get_sparse_core_info()
    nc, ns = info.num_cores, info.num_subcores
    nw = nc * ns
    n_tok = idx_flat.shape[0]
    d = table.shape[1]
    per_w = n_tok // nw
    chunk = 512
    n_chunks = per_w // chunk
    mesh = plsc.VectorSubcoreMesh(core_axis_name="c", subcore_axis_name="s")

    @functools.partial(
        pl.kernel, mesh=mesh,
        out_type=jax.ShapeDtypeStruct((n_tok, d), _F32),
        scratch_types=[
            pltpu.VMEM((chunk,), jnp.int32),
            pltpu.VMEM((chunk, d), _F32),
            pltpu.VMEM_SHARED((table.shape[0], d), _F32),
            pltpu.SemaphoreType.DMA,
        ],
    )
    def k(table_hbm, idx_hbm, out_hbm, idx_v, rows_v, tab_sh, sem):
        sid = lax.axis_index("s")
        @pl.when(sid == 0)
        def _():
            pltpu.sync_copy(table_hbm, tab_sh)
        plsc.subcore_barrier()
        wid = sid * nc + lax.axis_index("c")
        base = wid * per_w
        for c in range(n_chunks):
            lo = base + c * chunk
            pltpu.sync_copy(idx_hbm.at[pl.ds(lo, chunk)], idx_v)
            pltpu.async_copy(tab_sh.at[idx_v], rows_v, sem).wait()
            pltpu.sync_copy(rows_v, out_hbm.at[pl.ds(lo, chunk)])

    return k(table, idx_flat)


def _encode_half(xh, w1m, b1, rw1m, rb1, w2m, b2, rw2m, rb2, codebook):
    Bh = xh.shape[0]
    N1 = 128 * 256
    N2 = 32 * 256
    xp = jnp.pad(xh, ((0, 0), (0, 0), (2, 2), (0, 0)))
    p1 = jnp.stack([xp[:, :, kh:kh + 509:4, :] for kh in range(8)], axis=1)
    p1 = p1[:, :, :, _H1_ORDER, :].reshape(Bh, 16, N1)

    full = lambda *s: pl.BlockSpec(s, lambda b: tuple(0 for _ in s))
    bvq, idx = pl.pallas_call(
        _fused_kernel,
        grid=(Bh,),
        in_specs=[
            pl.BlockSpec((1, 16, N1), lambda b: (b, 0, 0)),
            full(48, 16), full(48, 1), full(96, 48), full(96, 1),
            full(96, 384), full(96, 1), full(192, 96), full(192, 1),
            full(1024, 96), full(1024, 1),
        ],
        out_specs=[
            pl.BlockSpec((1, 96, N2), lambda b: (b, 0, 0)),
            pl.BlockSpec((1, 1, N2), lambda b: (b, 0, 0)),
        ],
        out_shape=[
            jax.ShapeDtypeStruct((Bh, 96, N2), _F32),
            jax.ShapeDtypeStruct((Bh, 1, N2), jnp.int32),
        ],
        scratch_shapes=[pltpu.VMEM((48, 4 * _SEC), _F32)],
    )(p1, w1m, b1, rw1m, rb1, w2m, b2, rw2m, rb2, codebook,
      jnp.arange(1024, dtype=_F32)[:, None])
    return bvq, idx.reshape(Bh, N2)


def kernel(x, conv1_w, conv1_b, rw1_w, rw1_b, conv2_w, conv2_b, rw2_w, rw2_b,
           codebook):
    B = x.shape[0]
    NSPLIT = 4
    Bh = B // NSPLIT
    N2 = 32 * 256

    w1m = jnp.transpose(conv1_w[:, :, :, 0], (0, 2, 1)).reshape(48, 16)
    w2m = jnp.transpose(conv2_w[:, :, :, 0], (0, 2, 1)).reshape(96, 384)
    args = (w1m, conv1_b[:, None], rw1_w[:, :, 0, 0], rw1_b[:, None],
            w2m, conv2_b[:, None], rw2_w[:, :, 0, 0], rw2_b[:, None], codebook)
    cb_pad = jnp.pad(codebook, ((0, 0), (0, 32)))

    halves = []
    for h in range(NSPLIT):
        bvq_h, idx_h = _encode_half(x[h * Bh:(h + 1) * Bh], *args)
        quant_h = _sc_gather(cb_pad, idx_h.reshape(Bh * N2))[:, :96]
        lat_h = jnp.transpose(quant_h.reshape(Bh, N2, 96), (0, 2, 1))
        halves.append((lat_h, idx_h, bvq_h))

    latent = jnp.concatenate([h[0] for h in halves]).reshape(B, 96, 32, 256)
    indices = jnp.concatenate([h[1] for h in halves])
    beforvq = jnp.concatenate([h[2] for h in halves]).reshape(B, 96, 32, 256)
    return (latent, indices, beforvq)


# speedup vs baseline: 6.9524x; 1.0009x over previous
import functools

import numpy as np

import jax
import jax.numpy as jnp
from jax import lax
from jax.experimental import pallas as pl
from jax.experimental.pallas import tpu as pltpu
from jax.experimental.pallas import tpu_sc as plsc


_F32 = jnp.float32
_SEC = 8448
_S2 = 4096
_NT2 = 2

_H1_ORDER = np.concatenate([
    np.arange(1, 33) * 4 - 2,
    np.arange(1, 33) * 4 - 1,
    np.arange(0, 32) * 4,
    np.arange(0, 32) * 4 + 1,
])
_PAD_CHUNKS = (0, _SEC, 2 * _SEC + 8192, 3 * _SEC + 8192)


def _fused_kernel(p1_ref, w1_ref, b1_ref, rw1_ref, rb1_ref,
                  w2_ref, b2_ref, rw2_ref, rb2_ref, cb_ref, kio_ref,
                  bvq_ref, idx_ref, gscr):
    for col in _PAD_CHUNKS:
        gscr[:, col:col + 256] = jnp.zeros((48, 256), _F32)
    p1s = p1_ref[0]
    y = jnp.dot(w1_ref[...], p1s, preferred_element_type=_F32) + b1_ref[...]
    y = jax.nn.gelu(y)
    z = jnp.dot(rw1_ref[...], y, preferred_element_type=_F32) + rb1_ref[...]
    g1 = z[:48] * jax.nn.sigmoid(z[48:])
    for r in range(4):
        off = _SEC * r + (256 if r < 2 else 0)
        gscr[:, off:off + 8192] = g1[:, 8192 * r:8192 * (r + 1)]

    cb = cb_ref[...]
    cbsq = jnp.sum(cb * cb, axis=1, keepdims=True)
    a_aug = jnp.concatenate([cb * -2.0, cbsq], axis=1)
    for t in range(_NT2):
        pieces = []
        for kh in range(8):
            q, r = kh // 4, kh % 4
            off = _SEC * r + 256 * q + _S2 * t
            pieces.append(gscr[:, off:off + _S2])
        p2 = jnp.concatenate(pieces, axis=0)
        y = jnp.dot(w2_ref[...], p2, preferred_element_type=_F32) + b2_ref[...]
        y = jax.nn.gelu(y)
        z = jnp.dot(rw2_ref[...], y, preferred_element_type=_F32) + rb2_ref[...]
        g = z[:96] * jax.nn.sigmoid(z[96:])
        bvq_ref[0, :, _S2 * t:_S2 * (t + 1)] = g

        g_aug = jnp.concatenate([g, jnp.ones((1, _S2), _F32)], axis=0)
        dist = jnp.dot(a_aug, g_aug, preferred_element_type=_F32)
        minval = jnp.min(dist, axis=0, keepdims=True)
        idxf = jnp.min(jnp.where(dist == minval, kio_ref[...], 2048.0), axis=0,
                       keepdims=True)
        idx_ref[0, :, _S2 * t:_S2 * (t + 1)] = idxf.astype(jnp.int32)


def _sc_gather(table, idx_flat):
    info = plsc.get_sparse_core_info()
    nc, ns = info.num_cores, info.num_subcores
    nw = nc * ns
    n_tok = idx_flat.shape[0]
    d = table.shape[1]
    per_w = n_tok // nw
    chunk = 512
    n_chunks = per_w // chunk
    mesh = plsc.VectorSubcoreMesh(core_axis_name="c", subcore_axis_name="s")

    @functools.partial(
        pl.kernel, mesh=mesh,
        out_type=jax.ShapeDtypeStruct((n_tok, d), _F32),
        scratch_types=[
            pltpu.VMEM((chunk,), jnp.int32),
            pltpu.VMEM((chunk, d), _F32),
            pltpu.VMEM_SHARED((table.shape[0], d), _F32),
            pltpu.SemaphoreType.DMA,
        ],
    )
    def k(table_hbm, idx_hbm, out_hbm, idx_v, rows_v, tab_sh, sem):
        sid = lax.axis_index("s")
        @pl.when(sid == 0)
        def _():
            pltpu.sync_copy(table_hbm, tab_sh)
        plsc.subcore_barrier()
        wid = sid * nc + lax.axis_index("c")
        base = wid * per_w
        for c in range(n_chunks):
            lo = base + c * chunk
            pltpu.sync_copy(idx_hbm.at[pl.ds(lo, chunk)], idx_v)
            pltpu.async_copy(tab_sh.at[idx_v], rows_v, sem).wait()
            pltpu.sync_copy(rows_v, out_hbm.at[pl.ds(lo, chunk)])

    return k(table, idx_flat)


def _encode_half(xh, w1m, b1, rw1m, rb1, w2m, b2, rw2m, rb2, codebook):
    Bh = xh.shape[0]
    N1 = 128 * 256
    N2 = 32 * 256
    xp = jnp.pad(xh, ((0, 0), (0, 0), (2, 2), (0, 0)))
    p1 = jnp.stack([xp[:, :, kh:kh + 509:4, :] for kh in range(8)], axis=1)
    p1 = p1[:, :, :, _H1_ORDER, :].reshape(Bh, 16, N1)

    full = lambda *s: pl.BlockSpec(s, lambda b: tuple(0 for _ in s))
    bvq, idx = pl.pallas_call(
        _fused_kernel,
        grid=(Bh,),
        in_specs=[
            pl.BlockSpec((1, 16, N1), lambda b: (b, 0, 0)),
            full(48, 16), full(48, 1), full(96, 48), full(96, 1),
            full(96, 384), full(96, 1), full(192, 96), full(192, 1),
            full(1024, 96), full(1024, 1),
        ],
        out_specs=[
            pl.BlockSpec((1, 96, N2), lambda b: (b, 0, 0)),
            pl.BlockSpec((1, 1, N2), lambda b: (b, 0, 0)),
        ],
        out_shape=[
            jax.ShapeDtypeStruct((Bh, 96, N2), _F32),
            jax.ShapeDtypeStruct((Bh, 1, N2), jnp.int32),
        ],
        scratch_shapes=[pltpu.VMEM((48, 4 * _SEC), _F32)],
    )(p1, w1m, b1, rw1m, rb1, w2m, b2, rw2m, rb2, codebook,
      jnp.arange(1024, dtype=_F32)[:, None])
    return bvq, idx.reshape(Bh, N2)


def kernel(x, conv1_w, conv1_b, rw1_w, rw1_b, conv2_w, conv2_b, rw2_w, rw2_b,
           codebook):
    B = x.shape[0]
    NSPLIT = 4
    Bh = B // NSPLIT
    N2 = 32 * 256

    w1m = jnp.transpose(conv1_w[:, :, :, 0], (0, 2, 1)).reshape(48, 16)
    w2m = jnp.transpose(conv2_w[:, :, :, 0], (0, 2, 1)).reshape(96, 384)
    args = (w1m, conv1_b[:, None], rw1_w[:, :, 0, 0], rw1_b[:, None],
            w2m, conv2_b[:, None], rw2_w[:, :, 0, 0], rw2_b[:, None], codebook)
    cb_pad = jnp.pad(codebook, ((0, 0), (0, 32)))

    halves = []
    for h in range(NSPLIT):
        bvq_h, idx_h = _encode_half(x[h * Bh:(h + 1) * Bh], *args)
        quant_h = _sc_gather(cb_pad, idx_h.reshape(Bh * N2))[:, :96]
        lat_h = jnp.transpose(quant_h.reshape(Bh, N2, 96), (0, 2, 1))
        halves.append((lat_h, idx_h, bvq_h))

    latent = jnp.concatenate([h[0] for h in halves]).reshape(B, 96, 32, 256)
    indices = jnp.concatenate([h[1] for h in halves])
    beforvq = jnp.concatenate([h[2] for h in halves]).reshape(B, 96, 32, 256)
    return (latent, indices, beforvq)
